# baseline (speedup 1.0000x reference)
import functools

import numpy as np
import jax
import jax.numpy as jnp
from jax import lax
from jax.experimental import pallas as pl
from jax.experimental.pallas import tpu as pltpu
from jax.experimental.pallas import tpu_sc as plsc

_F = 26
_E = 16
_B = 4096
_MLP_IN = _F * _E
_NC = 2
_NS = 16
_NW = _NC * _NS
_LPW = _B * _F // _NW
_CH = 128
_NCH = _LPW // _CH

_OFFSETS = np.arange(_F, dtype=np.int32) * 100000
_OFF_REP = np.tile(_OFFSETS, _LPW // _F)
_SEL = np.tile(np.eye(_E, dtype=np.float32), (_F, 1))


def _sc_gather_body(xf_hbm, off_hbm, table_hbm, fc16_hbm, outv_hbm, outf_hbm,
                    idx_v, off_v, fcrow_v, lane_v, rows_v, fcr16_v, fcs_v,
                    sem_t, sem_f):
    wid = lax.axis_index("s") * _NC + lax.axis_index("c")
    base = wid * _LPW
    pltpu.sync_copy(xf_hbm.at[pl.ds(base, _LPW)], idx_v)
    pltpu.sync_copy(off_hbm, off_v)

    def _add(i, carry):
        s = pl.ds(i * 16, 16)
        ix = idx_v[s] + off_v[s]
        idx_v[s] = ix
        fcrow_v[s] = lax.shift_right_logical(ix, 4)
        lane_v[s] = lax.bitwise_and(ix, 15)
        return carry

    lax.fori_loop(0, _LPW // 16, _add, 0)

    def _start(j):
        isl = pl.ds(j * _CH, _CH)
        pltpu.make_async_copy(
            table_hbm.at[idx_v.at[isl]], rows_v.at[isl], sem_t).start()
        pltpu.make_async_copy(
            fc16_hbm.at[fcrow_v.at[isl]], fcr16_v.at[isl], sem_f).start()

    def _wait_one():
        isl = pl.ds(0, _CH)
        pltpu.make_async_copy(
            table_hbm.at[idx_v.at[isl]], rows_v.at[isl], sem_t).wait()
        pltpu.make_async_copy(
            fc16_hbm.at[fcrow_v.at[isl]], fcr16_v.at[isl], sem_f).wait()

    _start(0)

    def _step(j, carry):
        _start(j + 1)
        _wait_one()
        return carry

    lax.fori_loop(0, _NCH - 1, _step, 0)
    _wait_one()

    def _sel(i, carry):
        s = pl.ds(i * 16, 16)
        rows = jnp.arange(16, dtype=jnp.int32) + i * 16
        fcs_v[s] = plsc.load_gather(fcr16_v, [rows, lane_v[s]])
        return carry

    lax.fori_loop(0, _LPW // 16, _sel, 0)

    pltpu.sync_copy(rows_v, outv_hbm.at[pl.ds(base, _LPW)])
    pltpu.sync_copy(fcs_v, outf_hbm.at[pl.ds(base, _LPW)])


@functools.lru_cache(maxsize=1)
def _sc_gather():
    return pl.kernel(
        _sc_gather_body,
        out_type=(jax.ShapeDtypeStruct((_B * _F, _E), jnp.float32),
                  jax.ShapeDtypeStruct((_B * _F,), jnp.float32)),
        mesh=plsc.VectorSubcoreMesh(core_axis_name="c", subcore_axis_name="s",
                                    num_cores=_NC, num_subcores=_NS),
        scratch_types=[
            pltpu.VMEM((_LPW,), jnp.int32),
            pltpu.VMEM((_LPW,), jnp.int32),
            pltpu.VMEM((_LPW,), jnp.int32),
            pltpu.VMEM((_LPW,), jnp.int32),
            pltpu.VMEM((_LPW, _E), jnp.float32),
            pltpu.VMEM((_LPW, _E), jnp.float32),
            pltpu.VMEM((_LPW,), jnp.float32),
            pltpu.SemaphoreType.DMA,
            pltpu.SemaphoreType.DMA,
        ],
        compiler_params=pltpu.CompilerParams(use_tc_tiling_on_sc=False,
                                             needs_layout_passes=False),
    )


def _tc_body(v_ref, fcg_ref, sel_ref, w1_ref, b1_ref, g1_ref, be1_ref,
             w2_ref, b2_ref, g2_ref, be2_ref, w3_ref, b3_ref, bias_ref, o_ref):
    v = v_ref[...]
    s = jnp.dot(v, sel_ref[...], preferred_element_type=jnp.float32)
    inter = 0.5 * (jnp.sum(s * s, axis=1, keepdims=True)
                   - jnp.sum(v * v, axis=1, keepdims=True))
    lin = jnp.sum(fcg_ref[...], axis=1, keepdims=True) + bias_ref[0, 0]

    h = jnp.dot(v, w1_ref[...], preferred_element_type=jnp.float32) + b1_ref[...]
    m = jnp.mean(h, axis=0, keepdims=True)
    var = jnp.mean((h - m) ** 2, axis=0, keepdims=True)
    h = jnp.maximum((h - m) / jnp.sqrt(var + 1e-5) * g1_ref[...] + be1_ref[...],
                    0.0)

    h = jnp.dot(h, w2_ref[...], preferred_element_type=jnp.float32) + b2_ref[...]
    m = jnp.mean(h, axis=0, keepdims=True)
    var = jnp.mean((h - m) ** 2, axis=0, keepdims=True)
    h = jnp.maximum((h - m) / jnp.sqrt(var + 1e-5) * g2_ref[...] + be2_ref[...],
                    0.0)

    y = jnp.dot(h, w3_ref[...], preferred_element_type=jnp.float32) + b3_ref[...]
    o_ref[...] = jax.nn.sigmoid(lin + inter + y)


def kernel(x, table, fc, bias, W1, b1, g1, be1, W2, b2, g2, be2, W3, b3):
    xf = x.reshape(-1)
    off = jnp.asarray(_OFF_REP)
    v_rows, fc_rows = _sc_gather()(xf, off, table, fc.reshape(-1, _E))
    v2d = v_rows.reshape(_B, _MLP_IN)
    fcg = fc_rows.reshape(_B, _F)
    sel = jnp.asarray(_SEL)
    out = pl.pallas_call(
        _tc_body,
        out_shape=jax.ShapeDtypeStruct((_B, 1), jnp.float32),
    )(v2d, fcg, sel,
      W1, b1.reshape(1, -1), g1.reshape(1, -1), be1.reshape(1, -1),
      W2, b2.reshape(1, -1), g2.reshape(1, -1), be2.reshape(1, -1),
      W3, b3.reshape(1, 1), bias.reshape(1, 1))
    return out[:, 0]


# speedup vs baseline: 1.1462x; 1.1462x over previous
import functools

import numpy as np
import jax
import jax.numpy as jnp
from jax import lax
from jax.experimental import pallas as pl
from jax.experimental.pallas import tpu as pltpu
from jax.experimental.pallas import tpu_sc as plsc

_F = 26
_E = 16
_B = 4096
_MLP_IN = _F * _E
_NC = 2
_NS = 16
_NW = _NC * _NS
_LPW = _B * _F // _NW
_BPW = _B // _NW
_TCH = 64
_NTCH = _LPW // _TCH
_CH = 128
_NCH = _LPW // _CH
_RW = 4096
_RG = (2600000 + _RW - 1) // _RW

_OFFSETS = np.arange(_F, dtype=np.int32) * 100000
_OFF_REP = np.tile(_OFFSETS, _LPW // _F)
_SEL = np.tile(np.eye(_E, dtype=np.float32), (_F, 1))


def _repack_body(tt_ref, o_ref):
    w = tt_ref[...]
    w3 = w.reshape(16, _RW // 16, 16)
    o_ref[...] = jnp.transpose(w3, (1, 0, 2)).reshape(_RW // 16, 256)


def _repack(tt):
    return pl.pallas_call(
        _repack_body,
        grid=(_RG,),
        in_specs=[pl.BlockSpec((16, _RW), lambda i: (0, i))],
        out_specs=pl.BlockSpec((_RW // 16, 256), lambda i: (i, 0)),
        out_shape=jax.ShapeDtypeStruct((162500, 256), jnp.float32),
    )(tt)


def _sc_table_body(xf_hbm, off_hbm, tp_hbm, outv_hbm,
                   idx_v, off_v, qrow_v, lane_v, gbuf_v, rows_v, sem_a, sem_b):
    wid = lax.axis_index("s") * _NC + lax.axis_index("c")
    base = wid * _LPW
    pltpu.sync_copy(xf_hbm.at[pl.ds(base, _LPW)], idx_v)
    pltpu.sync_copy(off_hbm, off_v)

    def _add(i, carry):
        s = pl.ds(i * 16, 16)
        ix = idx_v[s] + off_v[s]
        qrow_v[s] = lax.shift_right_logical(ix, 4)
        lane_v[s] = lax.bitwise_and(ix, 15)
        return carry

    lax.fori_loop(0, _LPW // 16, _add, 0)

    def _start(j, b):
        isl = pl.ds(j * _TCH, _TCH)
        @pl.when(b == 0)
        def _():
            pltpu.make_async_copy(
                tp_hbm.at[qrow_v.at[isl]], gbuf_v.at[0], sem_a).start()
        @pl.when(b != 0)
        def _():
            pltpu.make_async_copy(
                tp_hbm.at[qrow_v.at[isl]], gbuf_v.at[1], sem_b).start()

    def _wait_one(b):
        isl = pl.ds(0, _TCH)
        @pl.when(b == 0)
        def _():
            pltpu.make_async_copy(
                tp_hbm.at[qrow_v.at[isl]], gbuf_v.at[0], sem_a).wait()
        @pl.when(b != 0)
        def _():
            pltpu.make_async_copy(
                tp_hbm.at[qrow_v.at[isl]], gbuf_v.at[1], sem_b).wait()

    def _extract(j, b):
        for g in range(_TCH // 16):
            i16 = jnp.arange(16, dtype=jnp.int32) + g * 16
            m16 = lane_v[pl.ds(j * _TCH + g * 16, 16)]
            lk16 = i16 + j * _TCH
            smp16 = lax.div(lk16, 26)
            col16 = (lk16 - smp16 * 26) * 16
            b16 = jnp.full((16,), b, dtype=jnp.int32)
            for e in range(_E):
                vals = plsc.load_gather(
                    gbuf_v, [b16, i16, m16 + e * 16])
                plsc.store_scatter(rows_v, [smp16, col16 + e], vals)

    _start(0, 0)

    def _step(j, carry):
        b = lax.rem(j, 2)
        _start(j + 1, 1 - b)
        _wait_one(b)
        _extract(j, b)
        return carry

    lax.fori_loop(0, _NTCH - 1, _step, 0, unroll=False)
    _wait_one(lax.rem(_NTCH - 1, 2))
    _extract(_NTCH - 1, lax.rem(_NTCH - 1, 2))

    pltpu.sync_copy(rows_v, outv_hbm.at[pl.ds(wid * _BPW, _BPW)])


@functools.lru_cache(maxsize=1)
def _sc_table():
    return pl.kernel(
        _sc_table_body,
        out_type=jax.ShapeDtypeStruct((_B, _MLP_IN), jnp.float32),
        mesh=plsc.VectorSubcoreMesh(core_axis_name="c", subcore_axis_name="s",
                                    num_cores=_NC, num_subcores=_NS),
        scratch_types=[
            pltpu.VMEM((_LPW,), jnp.int32),
            pltpu.VMEM((_LPW,), jnp.int32),
            pltpu.VMEM((_LPW,), jnp.int32),
            pltpu.VMEM((_LPW,), jnp.int32),
            pltpu.VMEM((2, _TCH, 256), jnp.float32),
            pltpu.VMEM((_BPW, _MLP_IN), jnp.float32),
            pltpu.SemaphoreType.DMA,
            pltpu.SemaphoreType.DMA,
        ],
        compiler_params=pltpu.CompilerParams(use_tc_tiling_on_sc=True,
                                             needs_layout_passes=False),
    )


def _sc_fc_body(xf_hbm, off_hbm, fc16_hbm, outf_hbm,
                idx_v, off_v, fcrow_v, lane_v, fcr16_v, fcs_v, sem_f):
    wid = lax.axis_index("s") * _NC + lax.axis_index("c")
    base = wid * _LPW
    pltpu.sync_copy(xf_hbm.at[pl.ds(base, _LPW)], idx_v)
    pltpu.sync_copy(off_hbm, off_v)

    def _add(i, carry):
        s = pl.ds(i * 16, 16)
        ix = idx_v[s] + off_v[s]
        fcrow_v[s] = lax.shift_right_logical(ix, 4)
        lane_v[s] = lax.bitwise_and(ix, 15)
        return carry

    lax.fori_loop(0, _LPW // 16, _add, 0)

    def _start(j):
        isl = pl.ds(j * _CH, _CH)
        pltpu.make_async_copy(
            fc16_hbm.at[fcrow_v.at[isl]], fcr16_v.at[isl], sem_f).start()

    def _wait_one():
        isl = pl.ds(0, _CH)
        pltpu.make_async_copy(
            fc16_hbm.at[fcrow_v.at[isl]], fcr16_v.at[isl], sem_f).wait()

    _start(0)

    def _step(j, carry):
        _start(j + 1)
        _wait_one()
        return carry

    lax.fori_loop(0, _NCH - 1, _step, 0)
    _wait_one()

    def _sel(i, carry):
        s = pl.ds(i * 16, 16)
        rows = jnp.arange(16, dtype=jnp.int32) + i * 16
        fcs_v[s] = plsc.load_gather(fcr16_v, [rows, lane_v[s]])
        return carry

    lax.fori_loop(0, _LPW // 16, _sel, 0)

    pltpu.sync_copy(fcs_v, outf_hbm.at[pl.ds(base, _LPW)])


@functools.lru_cache(maxsize=1)
def _sc_fc():
    return pl.kernel(
        _sc_fc_body,
        out_type=jax.ShapeDtypeStruct((_B * _F,), jnp.float32),
        mesh=plsc.VectorSubcoreMesh(core_axis_name="c", subcore_axis_name="s",
                                    num_cores=_NC, num_subcores=_NS),
        scratch_types=[
            pltpu.VMEM((_LPW,), jnp.int32),
            pltpu.VMEM((_LPW,), jnp.int32),
            pltpu.VMEM((_LPW,), jnp.int32),
            pltpu.VMEM((_LPW,), jnp.int32),
            pltpu.VMEM((_LPW, _E), jnp.float32),
            pltpu.VMEM((_LPW,), jnp.float32),
            pltpu.SemaphoreType.DMA,
        ],
        compiler_params=pltpu.CompilerParams(use_tc_tiling_on_sc=False,
                                             needs_layout_passes=False),
    )


def _tc_body(v_ref, fcg_ref, sel_ref, w1_ref, b1_ref, g1_ref, be1_ref,
             w2_ref, b2_ref, g2_ref, be2_ref, w3_ref, b3_ref, bias_ref, o_ref):
    v = v_ref[...]
    s = jnp.dot(v, sel_ref[...], preferred_element_type=jnp.float32)
    inter = 0.5 * (jnp.sum(s * s, axis=1, keepdims=True)
                   - jnp.sum(v * v, axis=1, keepdims=True))
    lin = jnp.sum(fcg_ref[...], axis=1, keepdims=True) + bias_ref[0, 0]

    h = jnp.dot(v, w1_ref[...], preferred_element_type=jnp.float32) + b1_ref[...]
    m = jnp.mean(h, axis=0, keepdims=True)
    var = jnp.mean((h - m) ** 2, axis=0, keepdims=True)
    h = jnp.maximum((h - m) / jnp.sqrt(var + 1e-5) * g1_ref[...] + be1_ref[...],
                    0.0)

    h = jnp.dot(h, w2_ref[...], preferred_element_type=jnp.float32) + b2_ref[...]
    m = jnp.mean(h, axis=0, keepdims=True)
    var = jnp.mean((h - m) ** 2, axis=0, keepdims=True)
    h = jnp.maximum((h - m) / jnp.sqrt(var + 1e-5) * g2_ref[...] + be2_ref[...],
                    0.0)

    y = jnp.dot(h, w3_ref[...], preferred_element_type=jnp.float32) + b3_ref[...]
    o_ref[...] = jax.nn.sigmoid(lin + inter + y)


def kernel(x, table, fc, bias, W1, b1, g1, be1, W2, b2, g2, be2, W3, b3):
    xf = x.reshape(-1)
    off = jnp.asarray(_OFF_REP)
    tp = _repack(table.T)
    v2d = _sc_table()(xf, off, tp)
    fc_rows = _sc_fc()(xf, off, fc.reshape(-1, _E))
    fcg = fc_rows.reshape(_B, _F)
    sel = jnp.asarray(_SEL)
    out = pl.pallas_call(
        _tc_body,
        out_shape=jax.ShapeDtypeStruct((_B, 1), jnp.float32),
    )(v2d, fcg, sel,
      W1, b1.reshape(1, -1), g1.reshape(1, -1), be1.reshape(1, -1),
      W2, b2.reshape(1, -1), g2.reshape(1, -1), be2.reshape(1, -1),
      W3, b3.reshape(1, 1), bias.reshape(1, 1))
    return out[:, 0]


# speedup vs baseline: 1.3307x; 1.1609x over previous
import functools

import numpy as np
import jax
import jax.numpy as jnp
from jax import lax
from jax.experimental import pallas as pl
from jax.experimental.pallas import tpu as pltpu
from jax.experimental.pallas import tpu_sc as plsc

_F = 26
_E = 16
_B = 4096
_MLP_IN = _F * _E
_NC = 2
_NS = 16
_NW = _NC * _NS
_LPW = _B * _F // _NW
_BPW = _B // _NW
_TCH = 64
_NTCH = _LPW // _TCH
_CH = 128
_NCH = _LPW // _CH
_RW = 4096
_RG = (2600000 + _RW - 1) // _RW

_OFFSETS = np.arange(_F, dtype=np.int32) * 100000
_OFF_REP = np.tile(_OFFSETS, _LPW // _F)
_SEL = np.tile(np.eye(_E, dtype=np.float32), (_F, 1))


def _repack_body(tt_ref, o_ref):
    w = tt_ref[...]
    w3 = w.reshape(16, _RW // 16, 16)
    for e in range(16):
        o_ref[:, e * 16:(e + 1) * 16] = w3[e]


def _repack(tt):
    return pl.pallas_call(
        _repack_body,
        grid=(_RG,),
        in_specs=[pl.BlockSpec((16, _RW), lambda i: (0, i))],
        out_specs=pl.BlockSpec((_RW // 16, 256), lambda i: (i, 0)),
        out_shape=jax.ShapeDtypeStruct((162500, 256), jnp.float32),
    )(tt)


def _sc_table_body(xf_hbm, off_hbm, tp_hbm, outv_hbm,
                   idx_v, off_v, qrow_v, lane_v, gbuf_v, rows_v, sem_a, sem_b):
    wid = lax.axis_index("s") * _NC + lax.axis_index("c")
    base = wid * _LPW
    pltpu.sync_copy(xf_hbm.at[pl.ds(base, _LPW)], idx_v)
    pltpu.sync_copy(off_hbm, off_v)

    def _add(i, carry):
        s = pl.ds(i * 16, 16)
        ix = idx_v[s] + off_v[s]
        qrow_v[s] = lax.shift_right_logical(ix, 4)
        lane_v[s] = lax.bitwise_and(ix, 15)
        return carry

    lax.fori_loop(0, _LPW // 16, _add, 0)

    def _start(j, b):
        isl = pl.ds(j * _TCH, _TCH)
        @pl.when(b == 0)
        def _():
            pltpu.make_async_copy(
                tp_hbm.at[qrow_v.at[isl]], gbuf_v.at[0], sem_a).start()
        @pl.when(b != 0)
        def _():
            pltpu.make_async_copy(
                tp_hbm.at[qrow_v.at[isl]], gbuf_v.at[1], sem_b).start()

    def _wait_one(b):
        isl = pl.ds(0, _TCH)
        @pl.when(b == 0)
        def _():
            pltpu.make_async_copy(
                tp_hbm.at[qrow_v.at[isl]], gbuf_v.at[0], sem_a).wait()
        @pl.when(b != 0)
        def _():
            pltpu.make_async_copy(
                tp_hbm.at[qrow_v.at[isl]], gbuf_v.at[1], sem_b).wait()

    def _extract(j, b):
        for g in range(_TCH // 16):
            i16 = jnp.arange(16, dtype=jnp.int32) + g * 16
            m16 = lane_v[pl.ds(j * _TCH + g * 16, 16)]
            lk16 = i16 + j * _TCH
            smp16 = lax.div(lk16, 26)
            col16 = (lk16 - smp16 * 26) * 16
            b16 = jnp.full((16,), b, dtype=jnp.int32)
            for e in range(_E):
                vals = plsc.load_gather(
                    gbuf_v, [b16, i16, m16 + e * 16])
                plsc.store_scatter(rows_v, [smp16, col16 + e], vals)

    _start(0, 0)

    def _step(j, carry):
        b = lax.rem(j, 2)
        _start(j + 1, 1 - b)
        _wait_one(b)
        _extract(j, b)
        return carry

    lax.fori_loop(0, _NTCH - 1, _step, 0, unroll=False)
    _wait_one(lax.rem(_NTCH - 1, 2))
    _extract(_NTCH - 1, lax.rem(_NTCH - 1, 2))

    pltpu.sync_copy(rows_v, outv_hbm.at[pl.ds(wid * _BPW, _BPW)])


@functools.lru_cache(maxsize=1)
def _sc_table():
    return pl.kernel(
        _sc_table_body,
        out_type=jax.ShapeDtypeStruct((_B, _MLP_IN), jnp.float32),
        mesh=plsc.VectorSubcoreMesh(core_axis_name="c", subcore_axis_name="s",
                                    num_cores=_NC, num_subcores=_NS),
        scratch_types=[
            pltpu.VMEM((_LPW,), jnp.int32),
            pltpu.VMEM((_LPW,), jnp.int32),
            pltpu.VMEM((_LPW,), jnp.int32),
            pltpu.VMEM((_LPW,), jnp.int32),
            pltpu.VMEM((2, _TCH, 256), jnp.float32),
            pltpu.VMEM((_BPW, _MLP_IN), jnp.float32),
            pltpu.SemaphoreType.DMA,
            pltpu.SemaphoreType.DMA,
        ],
        compiler_params=pltpu.CompilerParams(use_tc_tiling_on_sc=True,
                                             needs_layout_passes=False),
    )


def _sc_fc_body(xf_hbm, off_hbm, fc16_hbm, outf_hbm,
                idx_v, off_v, fcrow_v, lane_v, fcr16_v, fcs_v, sem_f):
    wid = lax.axis_index("s") * _NC + lax.axis_index("c")
    base = wid * _LPW
    pltpu.sync_copy(xf_hbm.at[pl.ds(base, _LPW)], idx_v)
    pltpu.sync_copy(off_hbm, off_v)

    def _add(i, carry):
        s = pl.ds(i * 16, 16)
        ix = idx_v[s] + off_v[s]
        fcrow_v[s] = lax.shift_right_logical(ix, 4)
        lane_v[s] = lax.bitwise_and(ix, 15)
        return carry

    lax.fori_loop(0, _LPW // 16, _add, 0)

    def _start(j):
        isl = pl.ds(j * _CH, _CH)
        pltpu.make_async_copy(
            fc16_hbm.at[fcrow_v.at[isl]], fcr16_v.at[isl], sem_f).start()

    def _wait_one():
        isl = pl.ds(0, _CH)
        pltpu.make_async_copy(
            fc16_hbm.at[fcrow_v.at[isl]], fcr16_v.at[isl], sem_f).wait()

    _start(0)

    def _step(j, carry):
        _start(j + 1)
        _wait_one()
        return carry

    lax.fori_loop(0, _NCH - 1, _step, 0)
    _wait_one()

    def _sel(i, carry):
        s = pl.ds(i * 16, 16)
        rows = jnp.arange(16, dtype=jnp.int32) + i * 16
        fcs_v[s] = plsc.load_gather(fcr16_v, [rows, lane_v[s]])
        return carry

    lax.fori_loop(0, _LPW // 16, _sel, 0)

    pltpu.sync_copy(fcs_v, outf_hbm.at[pl.ds(base, _LPW)])


@functools.lru_cache(maxsize=1)
def _sc_fc():
    return pl.kernel(
        _sc_fc_body,
        out_type=jax.ShapeDtypeStruct((_B * _F,), jnp.float32),
        mesh=plsc.VectorSubcoreMesh(core_axis_name="c", subcore_axis_name="s",
                                    num_cores=_NC, num_subcores=_NS),
        scratch_types=[
            pltpu.VMEM((_LPW,), jnp.int32),
            pltpu.VMEM((_LPW,), jnp.int32),
            pltpu.VMEM((_LPW,), jnp.int32),
            pltpu.VMEM((_LPW,), jnp.int32),
            pltpu.VMEM((_LPW, _E), jnp.float32),
            pltpu.VMEM((_LPW,), jnp.float32),
            pltpu.SemaphoreType.DMA,
        ],
        compiler_params=pltpu.CompilerParams(use_tc_tiling_on_sc=False,
                                             needs_layout_passes=False),
    )


def _tc_body(v_ref, fcg_ref, sel_ref, w1_ref, b1_ref, g1_ref, be1_ref,
             w2_ref, b2_ref, g2_ref, be2_ref, w3_ref, b3_ref, bias_ref, o_ref):
    v = v_ref[...]
    s = jnp.dot(v, sel_ref[...], preferred_element_type=jnp.float32)
    inter = 0.5 * (jnp.sum(s * s, axis=1, keepdims=True)
                   - jnp.sum(v * v, axis=1, keepdims=True))
    lin = jnp.sum(fcg_ref[...], axis=1, keepdims=True) + bias_ref[0, 0]

    h = jnp.dot(v, w1_ref[...], preferred_element_type=jnp.float32) + b1_ref[...]
    m = jnp.mean(h, axis=0, keepdims=True)
    var = jnp.mean((h - m) ** 2, axis=0, keepdims=True)
    h = jnp.maximum((h - m) / jnp.sqrt(var + 1e-5) * g1_ref[...] + be1_ref[...],
                    0.0)

    h = jnp.dot(h, w2_ref[...], preferred_element_type=jnp.float32) + b2_ref[...]
    m = jnp.mean(h, axis=0, keepdims=True)
    var = jnp.mean((h - m) ** 2, axis=0, keepdims=True)
    h = jnp.maximum((h - m) / jnp.sqrt(var + 1e-5) * g2_ref[...] + be2_ref[...],
                    0.0)

    y = jnp.dot(h, w3_ref[...], preferred_element_type=jnp.float32) + b3_ref[...]
    o_ref[...] = jax.nn.sigmoid(lin + inter + y)


def kernel(x, table, fc, bias, W1, b1, g1, be1, W2, b2, g2, be2, W3, b3):
    xf = x.reshape(-1)
    off = jnp.asarray(_OFF_REP)
    tp = _repack(table.T)
    v2d = _sc_table()(xf, off, tp)
    fc_rows = _sc_fc()(xf, off, fc.reshape(-1, _E))
    fcg = fc_rows.reshape(_B, _F)
    sel = jnp.asarray(_SEL)
    out = pl.pallas_call(
        _tc_body,
        out_shape=jax.ShapeDtypeStruct((_B, 1), jnp.float32),
    )(v2d, fcg, sel,
      W1, b1.reshape(1, -1), g1.reshape(1, -1), be1.reshape(1, -1),
      W2, b2.reshape(1, -1), g2.reshape(1, -1), be2.reshape(1, -1),
      W3, b3.reshape(1, 1), bias.reshape(1, 1))
    return out[:, 0]
